# baseline (speedup 1.0000x reference)
import functools

import jax
import jax.numpy as jnp
from jax import lax
from jax.experimental import pallas as pl
from jax.experimental.pallas import tpu as pltpu
from jax.experimental.pallas import tpu_sc as plsc

B, N, F_DIM, H_DIM, E = 4, 2500, 128, 128, 80000
NV = B * N
NPAD = 10240
ET = B * E
NC, NS = 2, 16
NWORK = NC * NS
EPW = ET // NWORK
W_E = 80
NWIN = EPW // W_E
NBUF = 5
MBUF = 2
RPT = NPAD // NS


def _sc_mesh():
    return plsc.VectorSubcoreMesh(
        core_axis_name="c", subcore_axis_name="s", num_cores=NC, num_subcores=NS)


def _fix_dst_all(srcw, dstw):
    trash = NV + lax.iota(jnp.int32, 16)

    def row(w, carry):
        for k in range(W_E // 16):
            sl = pl.ds(k * 16, 16)
            sv = srcw[pl.ds(w * W_E + k * 16, 16)]
            dv = dstw[w, sl]
            bad = (sv == 0) & (dv == 0)
            dstw[w, sl] = jnp.where(bad, trash, dv)
        return carry

    lax.fori_loop(0, NWIN, row, 0)


def _sc_degree_body(src_hbm, dst_hbm, zdeg_hbm, upd_hbm, degpart_hbm,
                    dstfix_hbm, srcw, dstw, upd, degacc, gsem, *ssems):
    c = lax.axis_index("c")
    s = lax.axis_index("s")
    wid = c * NS + s
    rb = s * RPT
    pltpu.async_copy(src_hbm.at[wid], srcw, gsem)
    pltpu.async_copy(dst_hbm.at[wid], dstw, gsem)
    pltpu.sync_copy(zdeg_hbm.at[pl.ds(rb, RPT)], degacc.at[pl.ds(rb, RPT)])
    pltpu.make_async_copy(src_hbm.at[wid], srcw, gsem).wait()
    pltpu.make_async_copy(dst_hbm.at[wid], dstw, gsem).wait()
    pltpu.sync_copy(upd_hbm, upd)
    _fix_dst_all(srcw, dstw)
    pltpu.async_copy(dstw, dstfix_hbm.at[wid], gsem)
    plsc.subcore_barrier()

    def it(i, carry):
        for b in range(NBUF):
            w = i * NBUF + b
            @pl.when(w >= NBUF)
            def _():
                pltpu.make_async_copy(
                    upd, degpart_hbm.at[c, pl.ds(0, W_E)], ssems[b]).wait()
            pltpu.async_copy(upd, degacc.at[dstw.at[w]], ssems[b], add=True)
        return carry

    lax.fori_loop(0, NWIN // NBUF, it, 0)
    for b in range(NBUF):
        pltpu.make_async_copy(
            upd, degpart_hbm.at[c, pl.ds(0, W_E)], ssems[b]).wait()
    pltpu.make_async_copy(dstw, dstfix_hbm.at[wid], gsem).wait()
    plsc.subcore_barrier()
    pltpu.sync_copy(degacc.at[pl.ds(rb, RPT)],
                    degpart_hbm.at[c, pl.ds(rb, RPT)])


def _sc_degree(src3, dst3, zdeg, upd):
    return pl.kernel(
        _sc_degree_body,
        out_type=[jax.ShapeDtypeStruct((NC, NPAD), jnp.float32),
                  jax.ShapeDtypeStruct((NWORK, NWIN, W_E), jnp.int32)],
        mesh=_sc_mesh(),
        scratch_types=[
            pltpu.VMEM((EPW,), jnp.int32),
            pltpu.VMEM((NWIN, W_E), jnp.int32),
            pltpu.VMEM((W_E,), jnp.float32),
            pltpu.VMEM_SHARED((NPAD,), jnp.float32),
            pltpu.SemaphoreType.DMA,
        ] + [pltpu.SemaphoreType.DMA] * NBUF,
    )(src3, dst3, zdeg, upd)


def _sc_msg_body(g_hbm, src_hbm, dst_hbm, zrows_hbm, msgpart_hbm,
                 srcw, dstw, acc, rows, gsems, ssems):
    c = lax.axis_index("c")
    s = lax.axis_index("s")
    wid = c * NS + s
    rb = s * RPT
    pltpu.async_copy(src_hbm.at[wid], srcw, gsems[0])
    pltpu.async_copy(dst_hbm.at[wid], dstw, gsems[0])
    pltpu.async_copy(zrows_hbm.at[pl.ds(rb, RPT)], acc.at[pl.ds(rb, RPT)],
                     ssems[1])
    pltpu.make_async_copy(src_hbm.at[wid], srcw, gsems[0]).wait()
    pltpu.make_async_copy(dst_hbm.at[wid], dstw, gsems[0]).wait()
    pltpu.make_async_copy(zrows_hbm.at[pl.ds(rb, RPT)], acc.at[pl.ds(rb, RPT)],
                          ssems[1]).wait()
    plsc.subcore_barrier()

    pltpu.async_copy(g_hbm.at[srcw.at[pl.ds(0, W_E)]], rows.at[0], gsems[0])

    def slot(w, b):
        bn = (b + 1) % MBUF
        @pl.when(w + 1 < NWIN)
        def _():
            @pl.when(w >= 1)
            def _():
                pltpu.make_async_copy(
                    rows.at[bn], msgpart_hbm.at[c, pl.ds(0, W_E)],
                    ssems[bn]).wait()
            pltpu.async_copy(g_hbm.at[srcw.at[pl.ds((w + 1) * W_E, W_E)]],
                             rows.at[bn], gsems[bn])
        pltpu.make_async_copy(g_hbm.at[srcw.at[pl.ds(w * W_E, W_E)]],
                              rows.at[b], gsems[b]).wait()
        pltpu.async_copy(rows.at[b], acc.at[dstw.at[w]], ssems[b], add=True)

    def it(i, carry):
        for b in range(MBUF):
            slot(i * MBUF + b, b)
        return carry

    lax.fori_loop(0, NWIN // MBUF, it, 0)
    slot(NWIN - 1, (NWIN - 1) % MBUF)
    for b in range(MBUF):
        pltpu.make_async_copy(
            rows.at[b], msgpart_hbm.at[c, pl.ds(0, W_E)], ssems[b]).wait()
    plsc.subcore_barrier()
    pltpu.sync_copy(acc.at[pl.ds(rb, RPT)], msgpart_hbm.at[c, pl.ds(rb, RPT)])


def _sc_msg(g, src3, dst3, zrows):
    return pl.kernel(
        _sc_msg_body,
        out_type=jax.ShapeDtypeStruct((NC, NPAD, H_DIM), jnp.float32),
        mesh=_sc_mesh(),
        scratch_types=[
            pltpu.VMEM((EPW,), jnp.int32),
            pltpu.VMEM((NWIN, W_E), jnp.int32),
            pltpu.VMEM_SHARED((NPAD, H_DIM), jnp.float32),
            pltpu.VMEM((MBUF, W_E, H_DIM), jnp.float32),
            [pltpu.SemaphoreType.DMA] * MBUF,
            [pltpu.SemaphoreType.DMA] * MBUF,
        ],
    )(g, src3, dst3, zrows)


def _tc_b1_body(xf_ref, w1_ref, h_ref):
    h_ref[...] = jnp.dot(xf_ref[...], w1_ref[...],
                         preferred_element_type=jnp.float32)


def _tc_b1(xf, W1):
    return pl.pallas_call(
        _tc_b1_body,
        out_shape=jax.ShapeDtypeStruct((NPAD, H_DIM), jnp.float32),
    )(xf, W1)


def _tc_b2_body(degpart_ref, h_ref, g1_ref, dinv_ref):
    deg = lax.dot_general(
        degpart_ref[...], jnp.ones((NC, 1), jnp.float32),
        (((0,), (0,)), ((), ())), preferred_element_type=jnp.float32) + 1.0
    dinv = lax.rsqrt(deg)
    g1_ref[...] = dinv * h_ref[...]
    dinv_ref[...] = dinv


def _tc_b2(degpart, h):
    return pl.pallas_call(
        _tc_b2_body,
        out_shape=[
            jax.ShapeDtypeStruct((NPAD, H_DIM), jnp.float32),
            jax.ShapeDtypeStruct((NPAD, 1), jnp.float32),
        ],
    )(degpart, h)


def _tc_d_body(msgpart_ref, g1_ref, dinv_ref, b1_ref, w2_ref, g2_ref):
    m = msgpart_ref[0] + msgpart_ref[1] + g1_ref[...]
    h1 = jnp.maximum(dinv_ref[...] * m + b1_ref[...], 0.0)
    g2_ref[...] = dinv_ref[...] * jnp.dot(
        h1, w2_ref[...], preferred_element_type=jnp.float32)


def _tc_d(msgpart, g1, dinv, b1, W2):
    return pl.pallas_call(
        _tc_d_body,
        out_shape=jax.ShapeDtypeStruct((NPAD, H_DIM), jnp.float32),
    )(msgpart, g1, dinv, b1, W2)


def _tc_e_body(msgpart_ref, g2_ref, dinv_ref, b2_ref, out_ref):
    m = msgpart_ref[0] + msgpart_ref[1] + g2_ref[...]
    h2 = jnp.maximum(dinv_ref[...] * m + b2_ref[...], 0.0)
    rows = lax.broadcasted_iota(jnp.int32, (B, NPAD), 0)
    cols = lax.broadcasted_iota(jnp.int32, (B, NPAD), 1)
    pool = jnp.where((cols // N == rows) & (cols < NV), 1.0 / N, 0.0)
    out_ref[...] = lax.dot_general(
        pool, h2, (((1,), (0,)), ((), ())),
        preferred_element_type=jnp.float32)


def _tc_e(msgpart, g2, dinv, b2):
    return pl.pallas_call(
        _tc_e_body,
        out_shape=jax.ShapeDtypeStruct((B, H_DIM), jnp.float32),
    )(msgpart, g2, dinv, b2)


def kernel(x, edge_index, batch, W1, b1, W2, b2):
    offsets = (jnp.arange(B, dtype=edge_index.dtype) * N)[:, None, None]
    ei = edge_index + offsets
    ei = jnp.transpose(ei, (1, 0, 2)).reshape(2, -1)
    src2 = ei[0].reshape(NWORK, EPW)
    dst3 = ei[1].reshape(NWORK, NWIN, W_E)
    xf = jnp.pad(x.reshape(NV, F_DIM), ((0, NPAD - NV), (0, 0)))
    zrows = jnp.zeros((NPAD, H_DIM), jnp.float32)
    zdeg = jnp.zeros((NPAD,), jnp.float32)
    upd = jnp.ones((W_E,), jnp.float32)
    b1r = b1.reshape(1, H_DIM)
    b2r = b2.reshape(1, H_DIM)

    degpart, dstfix = _sc_degree(src2, dst3, zdeg, upd)
    h0 = _tc_b1(xf, W1)
    g1, dinv = _tc_b2(degpart, h0)
    msg1 = _sc_msg(g1, src2, dstfix, zrows)
    g2 = _tc_d(msg1, g1, dinv, b1r, W2)
    msg2 = _sc_msg(g2, src2, dstfix, zrows)
    return _tc_e(msg2, g2, dinv, b2r)


# speedup vs baseline: 1.0420x; 1.0420x over previous
import functools

import jax
import jax.numpy as jnp
from jax import lax
from jax.experimental import pallas as pl
from jax.experimental.pallas import tpu as pltpu
from jax.experimental.pallas import tpu_sc as plsc

B, N, F_DIM, H_DIM, E = 4, 2500, 128, 128, 80000
NV = B * N
NPAD = 10240
ET = B * E
NC, NS = 2, 16
NWORK = NC * NS
EPW = ET // NWORK
W_E = 80
NWIN = EPW // W_E
NBUF = 5
MBUF = 2
RPT = NPAD // NS


def _sc_mesh():
    return plsc.VectorSubcoreMesh(
        core_axis_name="c", subcore_axis_name="s", num_cores=NC, num_subcores=NS)


def _fix_all(srcw, dstw, off):
    trash = NV + lax.iota(jnp.int32, 16)

    def row(w, carry):
        for k in range(W_E // 16):
            sl = pl.ds(k * 16, 16)
            fsl = pl.ds(w * W_E + k * 16, 16)
            sv = srcw[fsl] + off
            dv = dstw[w, sl] + off
            bad = (sv == 0) & (dv == 0)
            srcw[fsl] = sv
            dstw[w, sl] = jnp.where(bad, trash, dv)
        return carry

    lax.fori_loop(0, NWIN, row, 0)


def _sc_degree_body(ei_hbm, eir_hbm, zdeg_hbm, upd_hbm, degpart_hbm,
                    srcfix_hbm, dstfix_hbm, srcw, dstw, upd, degacc,
                    gsem, *ssems):
    c = lax.axis_index("c")
    s = lax.axis_index("s")
    wid = c * NS + s
    g = wid // (NWORK // B)
    part = wid % (NWORK // B)
    sbase = g * 2 * E + part * EPW
    pltpu.async_copy(ei_hbm.at[pl.ds(sbase, EPW)], srcw, gsem)
    pltpu.async_copy(eir_hbm.at[g, 1, part], dstw, gsem)
    rb = s * RPT
    pltpu.sync_copy(zdeg_hbm.at[pl.ds(rb, RPT)], degacc.at[pl.ds(rb, RPT)])
    pltpu.make_async_copy(ei_hbm.at[pl.ds(sbase, EPW)], srcw, gsem).wait()
    pltpu.make_async_copy(eir_hbm.at[g, 1, part], dstw, gsem).wait()
    pltpu.sync_copy(upd_hbm, upd)
    _fix_all(srcw, dstw, g * N)
    pltpu.async_copy(dstw, dstfix_hbm.at[wid], gsem)
    pltpu.async_copy(srcw, srcfix_hbm.at[wid], gsem)
    plsc.subcore_barrier()

    def it(i, carry):
        for b in range(NBUF):
            w = i * NBUF + b
            @pl.when(w >= NBUF)
            def _():
                pltpu.make_async_copy(
                    upd, degpart_hbm.at[c, pl.ds(0, W_E)], ssems[b]).wait()
            pltpu.async_copy(upd, degacc.at[dstw.at[w]], ssems[b], add=True)
        return carry

    lax.fori_loop(0, NWIN // NBUF, it, 0)
    for b in range(NBUF):
        pltpu.make_async_copy(
            upd, degpart_hbm.at[c, pl.ds(0, W_E)], ssems[b]).wait()
    pltpu.make_async_copy(dstw, dstfix_hbm.at[wid], gsem).wait()
    pltpu.make_async_copy(srcw, srcfix_hbm.at[wid], gsem).wait()
    plsc.subcore_barrier()
    pltpu.sync_copy(degacc.at[pl.ds(rb, RPT)],
                    degpart_hbm.at[c, pl.ds(rb, RPT)])


def _sc_degree(ei, eir, zdeg, upd):
    return pl.kernel(
        _sc_degree_body,
        out_type=[jax.ShapeDtypeStruct((NC, NPAD), jnp.float32),
                  jax.ShapeDtypeStruct((NWORK, EPW), jnp.int32),
                  jax.ShapeDtypeStruct((NWORK, NWIN, W_E), jnp.int32)],
        mesh=_sc_mesh(),
        scratch_types=[
            pltpu.VMEM((EPW,), jnp.int32),
            pltpu.VMEM((NWIN, W_E), jnp.int32),
            pltpu.VMEM((W_E,), jnp.float32),
            pltpu.VMEM_SHARED((NPAD,), jnp.float32),
            pltpu.SemaphoreType.DMA,
        ] + [pltpu.SemaphoreType.DMA] * NBUF,
    )(ei, eir, zdeg, upd)


def _sc_msg_body(g_hbm, src_hbm, dst_hbm, zrows_hbm, msgpart_hbm,
                 srcw, dstw, acc, rows, gsems, ssems):
    c = lax.axis_index("c")
    s = lax.axis_index("s")
    wid = c * NS + s
    rb = s * RPT
    pltpu.async_copy(src_hbm.at[wid], srcw, gsems[0])
    pltpu.async_copy(dst_hbm.at[wid], dstw, gsems[0])
    pltpu.async_copy(zrows_hbm.at[pl.ds(rb, RPT)], acc.at[pl.ds(rb, RPT)],
                     ssems[1])
    pltpu.make_async_copy(src_hbm.at[wid], srcw, gsems[0]).wait()
    pltpu.make_async_copy(dst_hbm.at[wid], dstw, gsems[0]).wait()
    pltpu.make_async_copy(zrows_hbm.at[pl.ds(rb, RPT)], acc.at[pl.ds(rb, RPT)],
                          ssems[1]).wait()
    plsc.subcore_barrier()

    pltpu.async_copy(g_hbm.at[srcw.at[pl.ds(0, W_E)]], rows.at[0], gsems[0])

    def slot(w, b):
        bn = (b + 1) % MBUF
        @pl.when(w + 1 < NWIN)
        def _():
            @pl.when(w >= 1)
            def _():
                pltpu.make_async_copy(
                    rows.at[bn], msgpart_hbm.at[c, pl.ds(0, W_E)],
                    ssems[bn]).wait()
            pltpu.async_copy(g_hbm.at[srcw.at[pl.ds((w + 1) * W_E, W_E)]],
                             rows.at[bn], gsems[bn])
        pltpu.make_async_copy(g_hbm.at[srcw.at[pl.ds(w * W_E, W_E)]],
                              rows.at[b], gsems[b]).wait()
        pltpu.async_copy(rows.at[b], acc.at[dstw.at[w]], ssems[b], add=True)

    def it(i, carry):
        for b in range(MBUF):
            slot(i * MBUF + b, b)
        return carry

    lax.fori_loop(0, NWIN // MBUF, it, 0)
    slot(NWIN - 1, (NWIN - 1) % MBUF)
    for b in range(MBUF):
        pltpu.make_async_copy(
            rows.at[b], msgpart_hbm.at[c, pl.ds(0, W_E)], ssems[b]).wait()
    plsc.subcore_barrier()
    pltpu.sync_copy(acc.at[pl.ds(rb, RPT)], msgpart_hbm.at[c, pl.ds(rb, RPT)])


def _sc_msg(g, src3, dst3, zrows):
    return pl.kernel(
        _sc_msg_body,
        out_type=jax.ShapeDtypeStruct((NC, NPAD, H_DIM), jnp.float32),
        mesh=_sc_mesh(),
        scratch_types=[
            pltpu.VMEM((EPW,), jnp.int32),
            pltpu.VMEM((NWIN, W_E), jnp.int32),
            pltpu.VMEM_SHARED((NPAD, H_DIM), jnp.float32),
            pltpu.VMEM((MBUF, W_E, H_DIM), jnp.float32),
            [pltpu.SemaphoreType.DMA] * MBUF,
            [pltpu.SemaphoreType.DMA] * MBUF,
        ],
    )(g, src3, dst3, zrows)


def _tc_b1_body(xf_ref, w1_ref, h_ref):
    h_ref[...] = jnp.dot(xf_ref[...], w1_ref[...],
                         preferred_element_type=jnp.float32)


def _tc_b1(xf, W1):
    return pl.pallas_call(
        _tc_b1_body,
        out_shape=jax.ShapeDtypeStruct((NPAD, H_DIM), jnp.float32),
    )(xf, W1)


def _tc_b2_body(degpart_ref, h_ref, g1_ref, dinv_ref):
    deg = lax.dot_general(
        degpart_ref[...], jnp.ones((NC, 1), jnp.float32),
        (((0,), (0,)), ((), ())), preferred_element_type=jnp.float32) + 1.0
    dinv = lax.rsqrt(deg)
    g1_ref[...] = dinv * h_ref[...]
    dinv_ref[...] = dinv


def _tc_b2(degpart, h):
    return pl.pallas_call(
        _tc_b2_body,
        out_shape=[
            jax.ShapeDtypeStruct((NPAD, H_DIM), jnp.float32),
            jax.ShapeDtypeStruct((NPAD, 1), jnp.float32),
        ],
    )(degpart, h)


def _tc_d_body(msgpart_ref, g1_ref, dinv_ref, b1_ref, w2_ref, g2_ref):
    m = msgpart_ref[0] + msgpart_ref[1] + g1_ref[...]
    h1 = jnp.maximum(dinv_ref[...] * m + b1_ref[...], 0.0)
    g2_ref[...] = dinv_ref[...] * jnp.dot(
        h1, w2_ref[...], preferred_element_type=jnp.float32)


def _tc_d(msgpart, g1, dinv, b1, W2):
    return pl.pallas_call(
        _tc_d_body,
        out_shape=jax.ShapeDtypeStruct((NPAD, H_DIM), jnp.float32),
    )(msgpart, g1, dinv, b1, W2)


def _tc_e_body(msgpart_ref, g2_ref, dinv_ref, b2_ref, out_ref):
    m = msgpart_ref[0] + msgpart_ref[1] + g2_ref[...]
    h2 = jnp.maximum(dinv_ref[...] * m + b2_ref[...], 0.0)
    rows = lax.broadcasted_iota(jnp.int32, (B, NPAD), 0)
    cols = lax.broadcasted_iota(jnp.int32, (B, NPAD), 1)
    pool = jnp.where((cols // N == rows) & (cols < NV), 1.0 / N, 0.0)
    out_ref[...] = lax.dot_general(
        pool, h2, (((1,), (0,)), ((), ())),
        preferred_element_type=jnp.float32)


def _tc_e(msgpart, g2, dinv, b2):
    return pl.pallas_call(
        _tc_e_body,
        out_shape=jax.ShapeDtypeStruct((B, H_DIM), jnp.float32),
    )(msgpart, g2, dinv, b2)


def kernel(x, edge_index, batch, W1, b1, W2, b2):
    ei1 = edge_index.reshape(-1)
    eir = edge_index.reshape(B, 2, NWORK // B, NWIN, W_E)
    xf = jnp.pad(x.reshape(NV, F_DIM), ((0, NPAD - NV), (0, 0)))
    zrows = jnp.zeros((NPAD, H_DIM), jnp.float32)
    zdeg = jnp.zeros((NPAD,), jnp.float32)
    upd = jnp.ones((W_E,), jnp.float32)
    b1r = b1.reshape(1, H_DIM)
    b2r = b2.reshape(1, H_DIM)

    degpart, srcfix, dstfix = _sc_degree(ei1, eir, zdeg, upd)
    h0 = _tc_b1(xf, W1)
    g1, dinv = _tc_b2(degpart, h0)
    msg1 = _sc_msg(g1, srcfix, dstfix, zrows)
    g2 = _tc_d(msg1, g1, dinv, b1r, W2)
    msg2 = _sc_msg(g2, srcfix, dstfix, zrows)
    return _tc_e(msg2, g2, dinv, b2r)


# speedup vs baseline: 1.0852x; 1.0414x over previous
import functools

import jax
import jax.numpy as jnp
from jax import lax
from jax.experimental import pallas as pl
from jax.experimental.pallas import tpu as pltpu
from jax.experimental.pallas import tpu_sc as plsc

B, N, F_DIM, H_DIM, E = 4, 2500, 128, 128, 80000
NV = B * N
NPAD = 10240
ET = B * E
NC, NS = 2, 16
NWORK = NC * NS
EPW = ET // NWORK
W_E = 80
NWIN = EPW // W_E
NBUF = 5
MBUF = 2
RPT = NPAD // NS


def _sc_mesh():
    return plsc.VectorSubcoreMesh(
        core_axis_name="c", subcore_axis_name="s", num_cores=NC, num_subcores=NS)


def _fix_all(srcw, dstflat, dstw, off):
    trash = NV + lax.iota(jnp.int32, 16)

    def row(w, carry):
        for k in range(W_E // 16):
            fsl = pl.ds(w * W_E + k * 16, 16)
            sv = srcw[fsl] + off
            dv = dstflat[fsl] + off
            bad = (sv == 0) & (dv == 0)
            srcw[fsl] = sv
            dstw[w, pl.ds(k * 16, 16)] = jnp.where(bad, trash, dv)
        return carry

    lax.fori_loop(0, NWIN, row, 0)


def _sc_degree_body(ei_hbm, zdeg_hbm, upd_hbm, degpart_hbm,
                    srcfix_hbm, dstfix_hbm, srcw, dstflat, dstw, upd, degacc,
                    gsem, *ssems):
    c = lax.axis_index("c")
    s = lax.axis_index("s")
    wid = c * NS + s
    g = wid // (NWORK // B)
    part = wid % (NWORK // B)
    sbase = g * 2 * E + part * EPW
    dbase = sbase + E
    pltpu.async_copy(ei_hbm.at[pl.ds(sbase, EPW)], srcw, gsem)
    pltpu.async_copy(ei_hbm.at[pl.ds(dbase, EPW)], dstflat, gsem)
    rb = s * RPT
    pltpu.sync_copy(zdeg_hbm.at[pl.ds(rb, RPT)], degacc.at[pl.ds(rb, RPT)])
    pltpu.make_async_copy(ei_hbm.at[pl.ds(sbase, EPW)], srcw, gsem).wait()
    pltpu.make_async_copy(ei_hbm.at[pl.ds(dbase, EPW)], dstflat, gsem).wait()
    pltpu.sync_copy(upd_hbm, upd)
    _fix_all(srcw, dstflat, dstw, g * N)
    pltpu.async_copy(dstw, dstfix_hbm.at[wid], gsem)
    pltpu.async_copy(srcw, srcfix_hbm.at[wid], gsem)
    plsc.subcore_barrier()

    def it(i, carry):
        for b in range(NBUF):
            w = i * NBUF + b
            @pl.when(w >= NBUF)
            def _():
                pltpu.make_async_copy(
                    upd, degpart_hbm.at[c, pl.ds(0, W_E)], ssems[b]).wait()
            pltpu.async_copy(upd, degacc.at[dstw.at[w]], ssems[b], add=True)
        return carry

    lax.fori_loop(0, NWIN // NBUF, it, 0)
    for b in range(NBUF):
        pltpu.make_async_copy(
            upd, degpart_hbm.at[c, pl.ds(0, W_E)], ssems[b]).wait()
    pltpu.make_async_copy(dstw, dstfix_hbm.at[wid], gsem).wait()
    pltpu.make_async_copy(srcw, srcfix_hbm.at[wid], gsem).wait()
    plsc.subcore_barrier()
    pltpu.sync_copy(degacc.at[pl.ds(rb, RPT)],
                    degpart_hbm.at[c, pl.ds(rb, RPT)])


def _sc_degree(ei, zdeg, upd):
    return pl.kernel(
        _sc_degree_body,
        out_type=[jax.ShapeDtypeStruct((NC, NPAD), jnp.float32),
                  jax.ShapeDtypeStruct((NWORK, EPW), jnp.int32),
                  jax.ShapeDtypeStruct((NWORK, NWIN, W_E), jnp.int32)],
        mesh=_sc_mesh(),
        scratch_types=[
            pltpu.VMEM((EPW,), jnp.int32),
            pltpu.VMEM((EPW,), jnp.int32),
            pltpu.VMEM((NWIN, W_E), jnp.int32),
            pltpu.VMEM((W_E,), jnp.float32),
            pltpu.VMEM_SHARED((NPAD,), jnp.float32),
            pltpu.SemaphoreType.DMA,
        ] + [pltpu.SemaphoreType.DMA] * NBUF,
    )(ei, zdeg, upd)


def _sc_msg_body(g_hbm, src_hbm, dst_hbm, msgpart_hbm,
                 srcw, dstw, acc, rows, gsems, ssems):
    c = lax.axis_index("c")
    s = lax.axis_index("s")
    wid = c * NS + s
    rb = s * RPT
    pltpu.async_copy(src_hbm.at[wid], srcw, gsems[0])
    pltpu.async_copy(dst_hbm.at[wid], dstw, gsems[0])
    @pl.when(c == 0)
    def _():
        pltpu.async_copy(g_hbm.at[pl.ds(rb, RPT)], acc.at[pl.ds(rb, RPT)],
                         ssems[1])

    @pl.when(c == 1)
    def _():
        z16 = jnp.zeros((16,), jnp.float32)

        def zrow(r, carry):
            for k in range(H_DIM // 16):
                rows[1, r, pl.ds(k * 16, 16)] = z16
            return carry

        lax.fori_loop(0, W_E, zrow, 0)
        for j in range(RPT // W_E):
            pltpu.sync_copy(rows.at[1], acc.at[pl.ds(rb + j * W_E, W_E)])
    pltpu.make_async_copy(src_hbm.at[wid], srcw, gsems[0]).wait()
    pltpu.make_async_copy(dst_hbm.at[wid], dstw, gsems[0]).wait()

    @pl.when(c == 0)
    def _():
        pltpu.make_async_copy(g_hbm.at[pl.ds(rb, RPT)], acc.at[pl.ds(rb, RPT)],
                              ssems[1]).wait()
    plsc.subcore_barrier()

    pltpu.async_copy(g_hbm.at[srcw.at[pl.ds(0, W_E)]], rows.at[0], gsems[0])

    def slot(w, b):
        bn = (b + 1) % MBUF
        @pl.when(w + 1 < NWIN)
        def _():
            @pl.when(w >= 1)
            def _():
                pltpu.make_async_copy(
                    rows.at[bn], msgpart_hbm.at[c, pl.ds(0, W_E)],
                    ssems[bn]).wait()
            pltpu.async_copy(g_hbm.at[srcw.at[pl.ds((w + 1) * W_E, W_E)]],
                             rows.at[bn], gsems[bn])
        pltpu.make_async_copy(g_hbm.at[srcw.at[pl.ds(w * W_E, W_E)]],
                              rows.at[b], gsems[b]).wait()
        pltpu.async_copy(rows.at[b], acc.at[dstw.at[w]], ssems[b], add=True)

    def it(i, carry):
        for b in range(MBUF):
            slot(i * MBUF + b, b)
        return carry

    lax.fori_loop(0, NWIN // MBUF, it, 0)
    slot(NWIN - 1, (NWIN - 1) % MBUF)
    for b in range(MBUF):
        pltpu.make_async_copy(
            rows.at[b], msgpart_hbm.at[c, pl.ds(0, W_E)], ssems[b]).wait()
    plsc.subcore_barrier()
    pltpu.sync_copy(acc.at[pl.ds(rb, RPT)], msgpart_hbm.at[c, pl.ds(rb, RPT)])


def _sc_msg(g, src3, dst3):
    return pl.kernel(
        _sc_msg_body,
        out_type=jax.ShapeDtypeStruct((NC, NPAD, H_DIM), jnp.float32),
        mesh=_sc_mesh(),
        scratch_types=[
            pltpu.VMEM((EPW,), jnp.int32),
            pltpu.VMEM((NWIN, W_E), jnp.int32),
            pltpu.VMEM_SHARED((NPAD, H_DIM), jnp.float32),
            pltpu.VMEM((MBUF, W_E, H_DIM), jnp.float32),
            [pltpu.SemaphoreType.DMA] * MBUF,
            [pltpu.SemaphoreType.DMA] * MBUF,
        ],
    )(g, src3, dst3)


def _tc_b1_body(xf_ref, w1_ref, h_ref):
    h_ref[...] = jnp.dot(xf_ref[...], w1_ref[...],
                         preferred_element_type=jnp.float32)


def _tc_b1(xf, W1):
    return pl.pallas_call(
        _tc_b1_body,
        out_shape=jax.ShapeDtypeStruct((NPAD, H_DIM), jnp.float32),
    )(xf, W1)


def _tc_b2_body(degpart_ref, h_ref, g1_ref, dinv_ref):
    deg = lax.dot_general(
        degpart_ref[...], jnp.ones((NC, 1), jnp.float32),
        (((0,), (0,)), ((), ())), preferred_element_type=jnp.float32) + 1.0
    dinv = lax.rsqrt(deg)
    g1_ref[...] = dinv * h_ref[...]
    dinv_ref[...] = dinv


def _tc_b2(degpart, h):
    return pl.pallas_call(
        _tc_b2_body,
        out_shape=[
            jax.ShapeDtypeStruct((NPAD, H_DIM), jnp.float32),
            jax.ShapeDtypeStruct((NPAD, 1), jnp.float32),
        ],
    )(degpart, h)


def _tc_d_body(msgpart_ref, dinv_ref, b1_ref, w2_ref, g2_ref):
    m = msgpart_ref[0] + msgpart_ref[1]
    h1 = jnp.maximum(dinv_ref[...] * m + b1_ref[...], 0.0)
    g2_ref[...] = dinv_ref[...] * jnp.dot(
        h1, w2_ref[...], preferred_element_type=jnp.float32)


def _tc_d(msgpart, dinv, b1, W2):
    return pl.pallas_call(
        _tc_d_body,
        out_shape=jax.ShapeDtypeStruct((NPAD, H_DIM), jnp.float32),
    )(msgpart, dinv, b1, W2)


def _tc_e_body(msgpart_ref, dinv_ref, b2_ref, out_ref):
    m = msgpart_ref[0] + msgpart_ref[1]
    h2 = jnp.maximum(dinv_ref[...] * m + b2_ref[...], 0.0)
    rows = lax.broadcasted_iota(jnp.int32, (B, NPAD), 0)
    cols = lax.broadcasted_iota(jnp.int32, (B, NPAD), 1)
    pool = jnp.where((cols // N == rows) & (cols < NV), 1.0 / N, 0.0)
    out_ref[...] = lax.dot_general(
        pool, h2, (((1,), (0,)), ((), ())),
        preferred_element_type=jnp.float32)


def _tc_e(msgpart, dinv, b2):
    return pl.pallas_call(
        _tc_e_body,
        out_shape=jax.ShapeDtypeStruct((B, H_DIM), jnp.float32),
    )(msgpart, dinv, b2)


def kernel(x, edge_index, batch, W1, b1, W2, b2):
    ei1 = edge_index.reshape(-1)
    xf = jnp.pad(x.reshape(NV, F_DIM), ((0, NPAD - NV), (0, 0)))
    zdeg = jnp.zeros((NPAD,), jnp.float32)
    upd = jnp.ones((W_E,), jnp.float32)
    b1r = b1.reshape(1, H_DIM)
    b2r = b2.reshape(1, H_DIM)

    degpart, srcfix, dstfix = _sc_degree(ei1, zdeg, upd)
    h0 = _tc_b1(xf, W1)
    g1, dinv = _tc_b2(degpart, h0)
    msg1 = _sc_msg(g1, srcfix, dstfix)
    g2 = _tc_d(msg1, dinv, b1r, W2)
    msg2 = _sc_msg(g2, srcfix, dstfix)
    return _tc_e(msg2, dinv, b2r)
